# Initial kernel scaffold; baseline (speedup 1.0000x reference)
import jax
import jax.numpy as jnp
from jax.experimental import pallas as pl


def kernel(x, edge_index, W1, b1, W2, b2):
    raise NotImplementedError("write your pallas kernel here")


# speedup vs baseline: 34.0375x; 34.0375x over previous
import functools

import jax
import jax.numpy as jnp
from jax import lax
from jax.experimental import pallas as pl
from jax.experimental.pallas import tpu as pltpu
from jax.experimental.pallas import tpu_sc as plsc

NC = 2
NS = 16
LANES = 16

_MESH = dict(core_axis_name="c", subcore_axis_name="s")
_SC_PARAMS = pltpu.CompilerParams(use_tc_tiling_on_sc=False)


def _pick_chunk(ept, cap):
    best = None
    for c in range(8, cap + 1, 8):
        if ept % c == 0:
            best = c
    return best


def _sc_degree(dst, n_pad, ch):
    e = dst.shape[0]
    tiles = NC * NS
    ept = e // tiles
    slc = n_pad // NS

    @functools.partial(
        pl.kernel,
        out_type=jax.ShapeDtypeStruct((NC * n_pad,), jnp.float32),
        mesh=plsc.VectorSubcoreMesh(**_MESH),
        compiler_params=_SC_PARAMS,
        scratch_types=[
            pltpu.VMEM((ch,), jnp.int32),
            pltpu.VMEM((ch,), jnp.float32),
            pltpu.VMEM((slc,), jnp.float32),
            pltpu.VMEM_SHARED((n_pad,), jnp.float32),
            pltpu.SemaphoreType.DMA,
        ],
    )
    def deg_kernel(dst_hbm, out_hbm, idx_v, ones_v, zero_v, acc_sh, sem):
        c = lax.axis_index("c")
        s = lax.axis_index("s")
        wid = c * NS + s

        @pl.loop(0, ch, step=LANES)
        def _(i):
            ones_v[pl.ds(i, LANES)] = jnp.full((LANES,), 1.0, jnp.float32)

        @pl.loop(0, slc, step=LANES)
        def _(i):
            zero_v[pl.ds(i, LANES)] = jnp.zeros((LANES,), jnp.float32)

        pltpu.sync_copy(zero_v, acc_sh.at[pl.ds(s * slc, slc)])
        plsc.subcore_barrier()

        @pl.loop(0, ept, step=ch)
        def _(g):
            pltpu.sync_copy(dst_hbm.at[pl.ds(wid * ept + g, ch)], idx_v)
            pltpu.sync_copy(ones_v, acc_sh.at[idx_v], add=True)

        plsc.subcore_barrier()
        pltpu.sync_copy(acc_sh.at[pl.ds(s * slc, slc)], zero_v)
        pltpu.sync_copy(zero_v, out_hbm.at[pl.ds(c * n_pad + s * slc, slc)])

    return deg_kernel(dst)


def _sc_aggregate(table, src, dst, n_pad, ch):
    e = src.shape[0]
    d = table.shape[1]
    tiles = NC * NS
    ept = e // tiles
    slc = n_pad // NS

    @functools.partial(
        pl.kernel,
        out_type=jax.ShapeDtypeStruct((NC, n_pad, d), jnp.float32),
        mesh=plsc.VectorSubcoreMesh(**_MESH),
        compiler_params=_SC_PARAMS,
        scratch_types=[
            pltpu.VMEM((ch,), jnp.int32),
            pltpu.VMEM((ch,), jnp.int32),
            pltpu.VMEM((ch, d), jnp.float32),
            pltpu.VMEM_SHARED((n_pad, d), jnp.float32),
            pltpu.SemaphoreType.DMA,
        ],
    )
    def agg_kernel(t_hbm, src_hbm, dst_hbm, out_hbm,
                   si_v, di_v, rows_v, acc_sh, sem):
        c = lax.axis_index("c")
        s = lax.axis_index("s")
        wid = c * NS + s

        @pl.loop(0, slc)
        def _(r):
            @pl.loop(0, d, step=LANES)
            def _(f):
                rows_v[r, pl.ds(f, LANES)] = jnp.zeros((LANES,), jnp.float32)

        pltpu.sync_copy(rows_v.at[pl.ds(0, slc)],
                        acc_sh.at[pl.ds(s * slc, slc)])
        plsc.subcore_barrier()

        @pl.loop(0, ept, step=ch)
        def _(g):
            base = wid * ept + g
            pltpu.sync_copy(src_hbm.at[pl.ds(base, ch)], si_v)
            pltpu.async_copy(t_hbm.at[si_v], rows_v, sem).wait()
            pltpu.sync_copy(dst_hbm.at[pl.ds(base, ch)], di_v)
            pltpu.sync_copy(rows_v, acc_sh.at[di_v], add=True)

        plsc.subcore_barrier()
        pltpu.sync_copy(acc_sh.at[pl.ds(s * slc, slc)],
                        rows_v.at[pl.ds(0, slc)])
        pltpu.sync_copy(rows_v.at[pl.ds(0, slc)],
                        out_hbm.at[c, pl.ds(s * slc, slc)])

    return agg_kernel(table, src, dst)


def _tc_matmul(a, w):
    n, k = a.shape
    m = w.shape[1]
    rb = 1000

    def body(a_ref, w_ref, o_ref):
        o_ref[...] = jnp.dot(a_ref[...], w_ref[...],
                             preferred_element_type=jnp.float32)

    return pl.pallas_call(
        body,
        grid=(n // rb,),
        in_specs=[pl.BlockSpec((rb, k), lambda i: (i, 0)),
                  pl.BlockSpec((k, m), lambda i: (0, 0))],
        out_specs=pl.BlockSpec((rb, m), lambda i: (i, 0)),
        out_shape=jax.ShapeDtypeStruct((n, m), jnp.float32),
    )(a, w)


def _tc_scale(xw, d0, d1):
    n, m = xw.shape
    rb = 1000

    def body(xw_ref, d0_ref, d1_ref, t_ref, dinv_ref):
        deg = d0_ref[...] + d1_ref[...] + 1.0
        dinv = lax.rsqrt(jnp.maximum(deg, 1.0))
        dinv_ref[...] = dinv
        t_ref[...] = xw_ref[...] * dinv

    return pl.pallas_call(
        body,
        grid=(n // rb,),
        in_specs=[pl.BlockSpec((rb, m), lambda i: (i, 0)),
                  pl.BlockSpec((rb, 1), lambda i: (i, 0)),
                  pl.BlockSpec((rb, 1), lambda i: (i, 0))],
        out_specs=[pl.BlockSpec((rb, m), lambda i: (i, 0)),
                   pl.BlockSpec((rb, 1), lambda i: (i, 0))],
        out_shape=[jax.ShapeDtypeStruct((n, m), jnp.float32),
                   jax.ShapeDtypeStruct((n, 1), jnp.float32)],
    )(xw, d0, d1)


def _tc_relu_scale(s_a, s_b, t, dinv, b):
    n, m = t.shape
    rb = 1000

    def body(sa_ref, sb_ref, t_ref, dinv_ref, b_ref, o_ref):
        dinv = dinv_ref[...]
        h = dinv * (sa_ref[...] + sb_ref[...] + t_ref[...]) + b_ref[...]
        o_ref[...] = dinv * jnp.maximum(h, 0.0)

    return pl.pallas_call(
        body,
        grid=(n // rb,),
        in_specs=[pl.BlockSpec((rb, m), lambda i: (i, 0)),
                  pl.BlockSpec((rb, m), lambda i: (i, 0)),
                  pl.BlockSpec((rb, m), lambda i: (i, 0)),
                  pl.BlockSpec((rb, 1), lambda i: (i, 0)),
                  pl.BlockSpec((1, m), lambda i: (0, 0))],
        out_specs=pl.BlockSpec((rb, m), lambda i: (i, 0)),
        out_shape=jax.ShapeDtypeStruct((n, m), jnp.float32),
    )(s_a, s_b, t, dinv, b)


def _tc_final(s_a, s_b, t, dinv, w2, b2):
    n, k = t.shape
    m = w2.shape[1]
    rb = 1000

    def body(sa_ref, sb_ref, t_ref, dinv_ref, w_ref, b_ref, o_ref):
        z = dinv_ref[...] * (sa_ref[...] + sb_ref[...] + t_ref[...])
        o_ref[...] = jnp.dot(z, w_ref[...],
                             preferred_element_type=jnp.float32) + b_ref[...]

    return pl.pallas_call(
        body,
        grid=(n // rb,),
        in_specs=[pl.BlockSpec((rb, k), lambda i: (i, 0)),
                  pl.BlockSpec((rb, k), lambda i: (i, 0)),
                  pl.BlockSpec((rb, k), lambda i: (i, 0)),
                  pl.BlockSpec((rb, 1), lambda i: (i, 0)),
                  pl.BlockSpec((k, m), lambda i: (0, 0)),
                  pl.BlockSpec((1, m), lambda i: (0, 0))],
        out_specs=pl.BlockSpec((rb, m), lambda i: (i, 0)),
        out_shape=jax.ShapeDtypeStruct((n, m), jnp.float32),
    )(s_a, s_b, t, dinv, w2, b2)


def kernel(x, edge_index, W1, b1, W2, b2):
    n = x.shape[0]
    e = edge_index.shape[1]
    d_hid = W1.shape[1]

    tiles = NC * NS
    assert e % tiles == 0
    ept = e // tiles
    ch_deg = _pick_chunk(ept, 2000)
    ch_agg = _pick_chunk(ept, 1000)
    n_pad = ((n + NS * 8 - 1) // (NS * 8)) * (NS * 8)

    src = edge_index[0]
    dst = edge_index[1]

    deg_parts = _sc_degree(dst, n_pad, ch_deg)
    xw = _tc_matmul(x, W1)

    d0 = deg_parts[:n].reshape(n, 1)
    d1 = deg_parts[n_pad:n_pad + n].reshape(n, 1)
    t1, dinv = _tc_scale(xw, d0, d1)

    s1 = _sc_aggregate(t1, src, dst, n_pad, ch_agg)
    t2 = _tc_relu_scale(s1[0, :n], s1[1, :n], t1, dinv,
                        b1.reshape(1, d_hid))

    s2 = _sc_aggregate(t2, src, dst, n_pad, ch_agg)
    out = _tc_final(s2[0, :n], s2[1, :n], t2, dinv, W2,
                    b2.reshape(1, W2.shape[1]))
    return out


# speedup vs baseline: 41.2517x; 1.2119x over previous
import functools

import jax
import jax.numpy as jnp
from jax import lax
from jax.experimental import pallas as pl
from jax.experimental.pallas import tpu as pltpu
from jax.experimental.pallas import tpu_sc as plsc

NC = 2
NS = 16
LANES = 16

_MESH = dict(core_axis_name="c", subcore_axis_name="s")
_SC_PARAMS = pltpu.CompilerParams(use_tc_tiling_on_sc=False)


def _pick_chunk(ept, cap):
    best = None
    for c in range(8, cap + 1, 8):
        if ept % c == 0:
            best = c
    return best


def _sc_degree(dst, n_pad, ch):
    e = dst.shape[0]
    tiles = NC * NS
    ept = e // tiles
    slc = n_pad // NS

    @functools.partial(
        pl.kernel,
        out_type=jax.ShapeDtypeStruct((NC * n_pad,), jnp.float32),
        mesh=plsc.VectorSubcoreMesh(**_MESH),
        compiler_params=_SC_PARAMS,
        scratch_types=[
            pltpu.VMEM((ch,), jnp.int32),
            pltpu.VMEM((ch,), jnp.float32),
            pltpu.VMEM((slc,), jnp.float32),
            pltpu.VMEM_SHARED((n_pad,), jnp.float32),
            pltpu.SemaphoreType.DMA,
        ],
    )
    def deg_kernel(dst_hbm, out_hbm, idx_v, ones_v, zero_v, acc_sh, sem):
        c = lax.axis_index("c")
        s = lax.axis_index("s")
        wid = c * NS + s

        @pl.loop(0, ch, step=LANES)
        def _(i):
            ones_v[pl.ds(i, LANES)] = jnp.full((LANES,), 1.0, jnp.float32)

        @pl.loop(0, slc, step=LANES)
        def _(i):
            zero_v[pl.ds(i, LANES)] = jnp.zeros((LANES,), jnp.float32)

        pltpu.sync_copy(zero_v, acc_sh.at[pl.ds(s * slc, slc)])
        plsc.subcore_barrier()

        @pl.loop(0, ept, step=ch)
        def _(g):
            pltpu.sync_copy(dst_hbm.at[pl.ds(wid * ept + g, ch)], idx_v)
            pltpu.sync_copy(ones_v, acc_sh.at[idx_v], add=True)

        plsc.subcore_barrier()
        pltpu.sync_copy(acc_sh.at[pl.ds(s * slc, slc)], zero_v)
        pltpu.sync_copy(zero_v, out_hbm.at[pl.ds(c * n_pad + s * slc, slc)])

    return deg_kernel(dst)


def _sc_aggregate(table, src, dst, n_pad, ch):
    e = src.shape[0]
    d = table.shape[1]
    tiles = NC * NS
    ept = e // tiles
    slc = n_pad // NS
    n_ch = ept // ch

    @functools.partial(
        pl.kernel,
        out_type=jax.ShapeDtypeStruct((NC, n_pad, d), jnp.float32),
        mesh=plsc.VectorSubcoreMesh(**_MESH),
        compiler_params=_SC_PARAMS,
        scratch_types=[
            pltpu.VMEM((ept,), jnp.int32),
            pltpu.VMEM((n_ch, ch), jnp.int32),
            pltpu.VMEM((ch, d), jnp.float32),
            pltpu.VMEM((ch, d), jnp.float32),
            pltpu.VMEM((slc // 2, d), jnp.float32),
            pltpu.VMEM_SHARED((n_pad, d), jnp.float32),
            pltpu.SemaphoreType.DMA,
            pltpu.SemaphoreType.DMA,
        ],
    )
    def agg_kernel(t_hbm, src_hbm, dst3_hbm, out_hbm,
                   si_v, di_v, rows0_v, rows1_v, stage_v, acc_sh, sem0, sem1):
        c = lax.axis_index("c")
        s = lax.axis_index("s")
        wid = c * NS + s
        rows = (rows0_v, rows1_v)
        sems = (sem0, sem1)

        pltpu.sync_copy(src_hbm.at[pl.ds(wid * ept, ept)], si_v)
        pltpu.sync_copy(dst3_hbm.at[wid], di_v)

        half = slc // 2

        @pl.loop(0, half)
        def _(r):
            @pl.loop(0, d, step=LANES)
            def _(f):
                stage_v[r, pl.ds(f, LANES)] = jnp.zeros((LANES,), jnp.float32)

        pltpu.sync_copy(stage_v, acc_sh.at[pl.ds(s * slc, half)])
        pltpu.sync_copy(stage_v, acc_sh.at[pl.ds(s * slc + half, half)])
        plsc.subcore_barrier()

        for b in range(2):
            pltpu.async_copy(t_hbm.at[si_v.at[pl.ds(b * ch, ch)]],
                             rows[b], sems[b])

        @pl.loop(0, n_ch, step=2)
        def _(g0):
            for b in range(2):
                g = g0 + b
                pltpu.make_async_copy(
                    t_hbm.at[si_v.at[pl.ds(0, ch)]], rows[b], sems[b]).wait()
                pltpu.sync_copy(rows[b], acc_sh.at[di_v.at[g]], add=True)

                @pl.when(g + 2 < n_ch)
                def _():
                    pltpu.async_copy(
                        t_hbm.at[si_v.at[pl.ds((g + 2) * ch, ch)]],
                        rows[b], sems[b])

        plsc.subcore_barrier()
        for k in range(2):
            pltpu.sync_copy(acc_sh.at[pl.ds(s * slc + k * half, half)],
                            stage_v)
            pltpu.sync_copy(stage_v,
                            out_hbm.at[c, pl.ds(s * slc + k * half, half)])

    return agg_kernel(table, src, dst.reshape(tiles, n_ch, ch))


def _tc_matmul(a, w):
    n, k = a.shape
    m = w.shape[1]
    rb = 1000

    def body(a_ref, w_ref, o_ref):
        o_ref[...] = jnp.dot(a_ref[...], w_ref[...],
                             preferred_element_type=jnp.float32)

    return pl.pallas_call(
        body,
        grid=(n // rb,),
        in_specs=[pl.BlockSpec((rb, k), lambda i: (i, 0)),
                  pl.BlockSpec((k, m), lambda i: (0, 0))],
        out_specs=pl.BlockSpec((rb, m), lambda i: (i, 0)),
        out_shape=jax.ShapeDtypeStruct((n, m), jnp.float32),
    )(a, w)


def _tc_scale(xw, d0, d1):
    n, m = xw.shape
    rb = 1000

    def body(xw_ref, d0_ref, d1_ref, t_ref, dinv_ref):
        deg = d0_ref[...] + d1_ref[...] + 1.0
        dinv = lax.rsqrt(jnp.maximum(deg, 1.0))
        dinv_ref[...] = dinv
        t_ref[...] = xw_ref[...] * dinv

    return pl.pallas_call(
        body,
        grid=(n // rb,),
        in_specs=[pl.BlockSpec((rb, m), lambda i: (i, 0)),
                  pl.BlockSpec((rb, 1), lambda i: (i, 0)),
                  pl.BlockSpec((rb, 1), lambda i: (i, 0))],
        out_specs=[pl.BlockSpec((rb, m), lambda i: (i, 0)),
                   pl.BlockSpec((rb, 1), lambda i: (i, 0))],
        out_shape=[jax.ShapeDtypeStruct((n, m), jnp.float32),
                   jax.ShapeDtypeStruct((n, 1), jnp.float32)],
    )(xw, d0, d1)


def _tc_relu_scale(s_a, s_b, t, dinv, b):
    n, m = t.shape
    rb = 1000

    def body(sa_ref, sb_ref, t_ref, dinv_ref, b_ref, o_ref):
        dinv = dinv_ref[...]
        h = dinv * (sa_ref[...] + sb_ref[...] + t_ref[...]) + b_ref[...]
        o_ref[...] = dinv * jnp.maximum(h, 0.0)

    return pl.pallas_call(
        body,
        grid=(n // rb,),
        in_specs=[pl.BlockSpec((rb, m), lambda i: (i, 0)),
                  pl.BlockSpec((rb, m), lambda i: (i, 0)),
                  pl.BlockSpec((rb, m), lambda i: (i, 0)),
                  pl.BlockSpec((rb, 1), lambda i: (i, 0)),
                  pl.BlockSpec((1, m), lambda i: (0, 0))],
        out_specs=pl.BlockSpec((rb, m), lambda i: (i, 0)),
        out_shape=jax.ShapeDtypeStruct((n, m), jnp.float32),
    )(s_a, s_b, t, dinv, b)


def _tc_final(s_a, s_b, t, dinv, w2, b2):
    n, k = t.shape
    m = w2.shape[1]
    rb = 1000

    def body(sa_ref, sb_ref, t_ref, dinv_ref, w_ref, b_ref, o_ref):
        z = dinv_ref[...] * (sa_ref[...] + sb_ref[...] + t_ref[...])
        o_ref[...] = jnp.dot(z, w_ref[...],
                             preferred_element_type=jnp.float32) + b_ref[...]

    return pl.pallas_call(
        body,
        grid=(n // rb,),
        in_specs=[pl.BlockSpec((rb, k), lambda i: (i, 0)),
                  pl.BlockSpec((rb, k), lambda i: (i, 0)),
                  pl.BlockSpec((rb, k), lambda i: (i, 0)),
                  pl.BlockSpec((rb, 1), lambda i: (i, 0)),
                  pl.BlockSpec((k, m), lambda i: (0, 0)),
                  pl.BlockSpec((1, m), lambda i: (0, 0))],
        out_specs=pl.BlockSpec((rb, m), lambda i: (i, 0)),
        out_shape=jax.ShapeDtypeStruct((n, m), jnp.float32),
    )(s_a, s_b, t, dinv, w2, b2)


def kernel(x, edge_index, W1, b1, W2, b2):
    n = x.shape[0]
    e = edge_index.shape[1]
    d_hid = W1.shape[1]

    tiles = NC * NS
    assert e % tiles == 0
    ept = e // tiles
    ch_deg = _pick_chunk(ept, 2000)
    ch_agg = _pick_chunk(ept, 200)
    n_pad = ((n + NS * 8 - 1) // (NS * 8)) * (NS * 8)

    src = edge_index[0]
    dst = edge_index[1]

    deg_parts = _sc_degree(dst, n_pad, ch_deg)
    xw = _tc_matmul(x, W1)

    d0 = deg_parts[:n].reshape(n, 1)
    d1 = deg_parts[n_pad:n_pad + n].reshape(n, 1)
    t1, dinv = _tc_scale(xw, d0, d1)

    s1 = _sc_aggregate(t1, src, dst, n_pad, ch_agg)
    t2 = _tc_relu_scale(s1[0, :n], s1[1, :n], t1, dinv,
                        b1.reshape(1, d_hid))

    s2 = _sc_aggregate(t2, src, dst, n_pad, ch_agg)
    out = _tc_final(s2[0, :n], s2[1, :n], t2, dinv, W2,
                    b2.reshape(1, W2.shape[1]))
    return out
